# Initial kernel scaffold; baseline (speedup 1.0000x reference)
import jax
import jax.numpy as jnp
from jax.experimental import pallas as pl


def kernel(x, edge_index, batch, W1_0, b1_0, W2_0, b2_0, gamma_0, beta_0, W1_1, b1_1, W2_1, b2_1, gamma_1, beta_1, W1_2, b1_2, W2_2, b2_2, gamma_2, beta_2, Wj, bj, Wc1, bc1, Wc2, bc2):
    raise NotImplementedError("write your pallas kernel here")


# speedup vs baseline: 5.6427x; 5.6427x over previous
import functools
import math

import jax
import jax.numpy as jnp
from jax import lax
from jax.experimental import pallas as pl
from jax.experimental.pallas import tpu as pltpu
from jax.experimental.pallas import tpu_sc as plsc

N_NODES = 10000
N_EDGES = 320000
G_GRAPHS = 64

TILES = 16
E_PER_TILE = N_EDGES // TILES
CHUNK = 125
N_CHUNKS = E_PER_TILE // CHUNK
STRIPE = 640
SUB = 128
GRP = 8


def _build_sc_agg(split_cols, interpret=False):
    width = 128
    n_chunks = N_EDGES // (TILES if split_cols else 2 * TILES) // CHUNK
    mesh = plsc.VectorSubcoreMesh(core_axis_name="c", subcore_axis_name="s",
                                  num_cores=2, num_subcores=TILES)

    def body(hL_hbm, hR_hbm, src_hbm, dst_hbm, outL_hbm, outR_hbm,
             idx_src, idx_dst, rows, wb, shared_agg, sem):
        cid = lax.axis_index("c")
        sid = lax.axis_index("s")

        n16 = width // 16

        def zero_wb(k, _):
            i = k // n16
            j = k - i * n16
            wb[i, pl.ds(j * 16, 16)] = jnp.zeros((16,), jnp.float32)
            return 0
        lax.fori_loop(0, SUB * n16, zero_wb, 0)

        def stripe_copy(to_shared, sub_fn):
            base = sid * STRIPE

            @pl.when(sid < TILES - 1)
            def _():
                for k in range(STRIPE // SUB):
                    sub_fn(base + k * SUB, SUB)

            @pl.when(sid == TILES - 1)
            def _():
                for k in range(3):
                    sub_fn(base + k * SUB, SUB)
                sub_fn(base + 3 * SUB, 16)

        for ci in range(2):
            h_ref = (hL_hbm, hR_hbm)[ci] if split_cols else hL_hbm
            out_ref = (outL_hbm, outR_hbm)[ci]
            tslice = sid if split_cols else ci * TILES + sid

            @pl.when(cid == ci)
            def _():
                def zero_sub(off, n):
                    pltpu.sync_copy(wb.at[pl.ds(0, n)],
                                    shared_agg.at[pl.ds(off, n)])
                stripe_copy(True, zero_sub)
                plsc.subcore_barrier()

                def group(g, _):
                    pltpu.sync_copy(
                        src_hbm.at[tslice, pl.ds(g * GRP, GRP)], idx_src)
                    pltpu.sync_copy(
                        dst_hbm.at[tslice, pl.ds(g * GRP, GRP)], idx_dst)
                    for j in range(GRP):
                        pltpu.async_copy(h_ref.at[idx_src.at[j]], rows,
                                         sem).wait()
                        pltpu.sync_copy(rows, shared_agg.at[idx_dst.at[j]],
                                        add=True)
                    return 0
                lax.fori_loop(0, n_chunks // GRP, group, 0)

                plsc.subcore_barrier()

                def out_sub(off, n):
                    pltpu.sync_copy(shared_agg.at[pl.ds(off, n)],
                                    wb.at[pl.ds(0, n)])
                    pltpu.sync_copy(wb.at[pl.ds(0, n)],
                                    out_ref.at[pl.ds(off, n)])
                stripe_copy(False, out_sub)

    def wrapped(hL, hR, src3, dst3):
        return pl.kernel(
            body,
            out_type=[jax.ShapeDtypeStruct((N_NODES, width), jnp.float32),
                      jax.ShapeDtypeStruct((N_NODES, width), jnp.float32)],
            mesh=mesh,
            scratch_types=[
                pltpu.VMEM((GRP, CHUNK), jnp.int32),
                pltpu.VMEM((GRP, CHUNK), jnp.int32),
                pltpu.VMEM((CHUNK, width), jnp.float32),
                pltpu.VMEM((SUB, width), jnp.float32),
                pltpu.VMEM_SHARED((N_NODES, width), jnp.float32),
                pltpu.SemaphoreType.DMA,
            ],
            interpret=interpret,
        )(hL, hR, src3, dst3)

    return wrapped


BR = 1000
N_BLOCKS = N_NODES // BR
BN_SCALE = 1.0 / math.sqrt(1.0 + 1e-5)


def _mlp_tail(z, W2, b2, gm, bt, outL, outR):
    a = jnp.maximum(z, 0.0)
    z2 = jnp.dot(a, W2[...], preferred_element_type=jnp.float32) + b2[...]
    h = z2 * (BN_SCALE * gm[...]) + bt[...]
    h = jnp.maximum(h, 0.0)
    outL[...] = h[:, :128]
    outR[...] = h[:, 128:]


def _mlp_body(hL, hR, aL, aR, W1, b1, W2, b2, gm, bt, outL, outR):
    w1 = W1[...]
    z = (jnp.dot(hL[...] + aL[...], w1[:128, :],
                 preferred_element_type=jnp.float32)
         + jnp.dot(hR[...] + aR[...], w1[128:, :],
                   preferred_element_type=jnp.float32)
         + b1[...])
    _mlp_tail(z, W2, b2, gm, bt, outL, outR)


def _mlp0_body(x, aA, aB, W1, b1, W2, b2, gm, bt, outL, outR):
    u = x[...] + aA[...] + aB[...]
    z = jnp.dot(u, W1[...], preferred_element_type=jnp.float32) + b1[...]
    _mlp_tail(z, W2, b2, gm, bt, outL, outR)


def _build_mlp(first, interpret=False):
    din = 128 if first else 256
    return pl.pallas_call(
        _mlp0_body if first else _mlp_body,
        grid=(N_BLOCKS,),
        in_specs=[
            pl.BlockSpec((BR, 128), lambda i: (i, 0)),
            pl.BlockSpec((BR, 128), lambda i: (i, 0)),
            pl.BlockSpec((BR, 128), lambda i: (i, 0)),
            pl.BlockSpec((BR, 128), lambda i: (i, 0)),
            pl.BlockSpec((din, 256), lambda i: (0, 0)),
            pl.BlockSpec((1, 256), lambda i: (0, 0)),
            pl.BlockSpec((256, 256), lambda i: (0, 0)),
            pl.BlockSpec((1, 256), lambda i: (0, 0)),
            pl.BlockSpec((1, 256), lambda i: (0, 0)),
            pl.BlockSpec((1, 256), lambda i: (0, 0)),
        ][0 if not first else 1:],
        out_specs=[
            pl.BlockSpec((BR, 128), lambda i: (i, 0)),
            pl.BlockSpec((BR, 128), lambda i: (i, 0)),
        ],
        out_shape=[jax.ShapeDtypeStruct((N_NODES, 128), jnp.float32),
                   jax.ShapeDtypeStruct((N_NODES, 128), jnp.float32)],
        interpret=interpret,
    )


def _final_body(h0L, h0R, h1L, h1R, h2L, h2R, batch, Wj, bj, Wc1, bc1,
                Wc2, bc2, out, acc, cnt):
    i = pl.program_id(0)

    @pl.when(i == 0)
    def _():
        acc[...] = jnp.zeros_like(acc)
        cnt[...] = jnp.zeros_like(cnt)

    hcat = jnp.concatenate(
        [h0L[...], h0R[...], h1L[...], h1R[...], h2L[...], h2R[...]], axis=1)
    seg = jax.lax.broadcasted_iota(jnp.int32, (1, G_GRAPHS), 1)
    onehot = (batch[...] == seg).astype(jnp.float32)
    acc[...] += lax.dot_general(onehot, hcat,
                                (((0,), (0,)), ((), ())),
                                preferred_element_type=jnp.float32)
    ones = jnp.ones((BR, 1), jnp.float32)
    cnt[...] += lax.dot_general(onehot, ones,
                                (((0,), (0,)), ((), ())),
                                preferred_element_type=jnp.float32)

    @pl.when(i == N_BLOCKS - 1)
    def _():
        pooled = (jnp.dot(acc[...], Wj[...],
                          preferred_element_type=jnp.float32)
                  + cnt[...] * bj[...])
        q = jnp.maximum(
            jnp.dot(pooled, Wc1[...], preferred_element_type=jnp.float32)
            + bc1[...], 0.0)
        out[...] = (jnp.dot(q, Wc2[...], preferred_element_type=jnp.float32)
                    + bc2[...])


def _build_final(interpret=False):
    hspec = pl.BlockSpec((BR, 128), lambda i: (i, 0))
    return pl.pallas_call(
        _final_body,
        grid=(N_BLOCKS,),
        in_specs=[
            hspec, hspec, hspec, hspec, hspec, hspec,
            pl.BlockSpec((BR, 1), lambda i: (i, 0)),
            pl.BlockSpec((768, 256), lambda i: (0, 0)),
            pl.BlockSpec((1, 256), lambda i: (0, 0)),
            pl.BlockSpec((256, 128), lambda i: (0, 0)),
            pl.BlockSpec((1, 128), lambda i: (0, 0)),
            pl.BlockSpec((128, 128), lambda i: (0, 0)),
            pl.BlockSpec((1, 128), lambda i: (0, 0)),
        ],
        out_specs=pl.BlockSpec((G_GRAPHS, 128), lambda i: (0, 0)),
        out_shape=jax.ShapeDtypeStruct((G_GRAPHS, 128), jnp.float32),
        scratch_shapes=[
            pltpu.VMEM((G_GRAPHS, 768), jnp.float32),
            pltpu.VMEM((G_GRAPHS, 1), jnp.float32),
        ],
        interpret=interpret,
    )


def _run(x, edge_index, batch, params, jump, head, *, interpret=False):
    src3s = edge_index[0].reshape(TILES, -1, CHUNK)
    dst3s = edge_index[1].reshape(TILES, -1, CHUNK)
    src3f = edge_index[0].reshape(2 * TILES, -1, CHUNK)
    dst3f = edge_index[1].reshape(2 * TILES, -1, CHUNK)
    batch2 = batch.reshape(N_NODES, 1)

    sc_full = _build_sc_agg(False, interpret=interpret)
    sc_split = _build_sc_agg(True, interpret=interpret)
    mlp0 = _build_mlp(True, interpret=interpret)
    mlp = _build_mlp(False, interpret=interpret)
    fin = _build_final(interpret=interpret)

    def row(v):
        return v.reshape(1, -1)

    halves = []
    hL = hR = None
    for li, (W1, b1, W2, b2, gm, bt) in enumerate(params):
        if li == 0:
            aA, aB = sc_full(x, x, src3f, dst3f)
            hL, hR = mlp0(x, aA, aB, W1, row(b1), W2, row(b2),
                          row(gm), row(bt))
        else:
            aL, aR = sc_split(hL, hR, src3s, dst3s)
            hL, hR = mlp(hL, hR, aL, aR, W1, row(b1), W2, row(b2),
                         row(gm), row(bt))
        halves.extend([hL, hR])

    Wj, bj = jump
    Wc1, bc1, Wc2, bc2 = head
    Wc2p = jnp.pad(Wc2, ((0, 0), (0, 128 - Wc2.shape[1])))
    bc2p = jnp.pad(bc2, (0, 128 - bc2.shape[0]))
    outp = fin(*halves, batch2, Wj, row(bj), Wc1, row(bc1), Wc2p, row(bc2p))
    return outp[:, :Wc2.shape[1]]


def kernel(x, edge_index, batch, W1_0, b1_0, W2_0, b2_0, gamma_0, beta_0,
           W1_1, b1_1, W2_1, b2_1, gamma_1, beta_1,
           W1_2, b1_2, W2_2, b2_2, gamma_2, beta_2,
           Wj, bj, Wc1, bc1, Wc2, bc2):
    params = [
        (W1_0, b1_0, W2_0, b2_0, gamma_0, beta_0),
        (W1_1, b1_1, W2_1, b2_1, gamma_1, beta_1),
        (W1_2, b1_2, W2_2, b2_2, gamma_2, beta_2),
    ]
    return _run(x, edge_index, batch, params, (Wj, bj), (Wc1, bc1, Wc2, bc2))


# speedup vs baseline: 8.2450x; 1.4612x over previous
import functools
import math

import jax
import jax.numpy as jnp
from jax import lax
from jax.experimental import pallas as pl
from jax.experimental.pallas import tpu as pltpu
from jax.experimental.pallas import tpu_sc as plsc

N_NODES = 10000
N_EDGES = 320000
G_GRAPHS = 64

TILES = 16
E_PER_TILE = N_EDGES // TILES
CHUNK = 125
N_CHUNKS = E_PER_TILE // CHUNK
STRIPE = 640
SUB = 128


def _build_sc_agg(split_cols, interpret=False):
    width = 128
    n_chunks = N_EDGES // (TILES if split_cols else 2 * TILES) // CHUNK
    mesh = plsc.VectorSubcoreMesh(core_axis_name="c", subcore_axis_name="s",
                                  num_cores=2, num_subcores=TILES)

    def body(hL_hbm, hR_hbm, src_hbm, dst_hbm, outL_hbm, outR_hbm,
             idx, rowsA, rowsB, wb, shared_agg, semg0, semg1, semi):
        semg = (semg0, semg1)
        cid = lax.axis_index("c")
        sid = lax.axis_index("s")

        n16 = width // 16

        def zero_wb(k, _):
            i = k // n16
            j = k - i * n16
            wb[i, pl.ds(j * 16, 16)] = jnp.zeros((16,), jnp.float32)
            return 0
        lax.fori_loop(0, SUB * n16, zero_wb, 0)

        def stripe_copy(to_shared, sub_fn):
            base = sid * STRIPE

            @pl.when(sid < TILES - 1)
            def _():
                for k in range(STRIPE // SUB):
                    sub_fn(base + k * SUB, SUB)

            @pl.when(sid == TILES - 1)
            def _():
                for k in range(3):
                    sub_fn(base + k * SUB, SUB)
                sub_fn(base + 3 * SUB, 16)

        for ci in range(2):
            h_ref = (hL_hbm, hR_hbm)[ci] if split_cols else hL_hbm
            out_ref = (outL_hbm, outR_hbm)[ci]
            tslice = sid if split_cols else ci * TILES + sid

            @pl.when(cid == ci)
            def _():
                def zero_sub(off, n):
                    pltpu.sync_copy(wb.at[pl.ds(0, n)],
                                    shared_agg.at[pl.ds(off, n)])
                stripe_copy(True, zero_sub)
                plsc.subcore_barrier()

                def idx_load(k, q, sem_):
                    return (pltpu.async_copy(src_hbm.at[tslice, k],
                                             idx.at[q], sem_),
                            pltpu.async_copy(dst_hbm.at[tslice, k],
                                             idx.at[2 + q], sem_))

                def idx_wait(k, q):
                    pltpu.make_async_copy(src_hbm.at[tslice, k],
                                          idx.at[q], semi).wait()
                    pltpu.make_async_copy(dst_hbm.at[tslice, k],
                                          idx.at[2 + q], semi).wait()

                for d in idx_load(0, 0, semi):
                    d.wait()
                pltpu.async_copy(h_ref.at[idx.at[0, 0]], rowsA, semg[0])
                idx_load(1, 1, semi)

                def step(k, q):
                    my_rows = (rowsA, rowsB)[q]
                    other = (rowsA, rowsB)[1 - q]

                    @pl.when(k + 1 < n_chunks)
                    def _():
                        idx_wait(k + 1, 1 - q)
                        pltpu.async_copy(h_ref.at[idx.at[1 - q, 0]], other,
                                         semg[1 - q])
                    pltpu.make_async_copy(h_ref.at[idx.at[q, 0]], my_rows,
                                          semg[q]).wait()
                    pltpu.sync_copy(my_rows, shared_agg.at[idx.at[2 + q, 0]],
                                    add=True)

                    @pl.when(k + 2 < n_chunks)
                    def _():
                        idx_load(k + 2, q, semi)

                def pair(p, _):
                    step(2 * p, 0)
                    step(2 * p + 1, 1)
                    return 0
                lax.fori_loop(0, n_chunks // 2, pair, 0)

                plsc.subcore_barrier()

                def out_sub(off, n):
                    pltpu.sync_copy(shared_agg.at[pl.ds(off, n)],
                                    wb.at[pl.ds(0, n)])
                    pltpu.sync_copy(wb.at[pl.ds(0, n)],
                                    out_ref.at[pl.ds(off, n)])
                stripe_copy(False, out_sub)

    def wrapped(hL, hR, src3, dst3):
        return pl.kernel(
            body,
            out_type=[jax.ShapeDtypeStruct((N_NODES, width), jnp.float32),
                      jax.ShapeDtypeStruct((N_NODES, width), jnp.float32)],
            mesh=mesh,
            scratch_types=[
                pltpu.VMEM((4, 1, CHUNK), jnp.int32),
                pltpu.VMEM((CHUNK, width), jnp.float32),
                pltpu.VMEM((CHUNK, width), jnp.float32),
                pltpu.VMEM((SUB, width), jnp.float32),
                pltpu.VMEM_SHARED((N_NODES, width), jnp.float32),
                pltpu.SemaphoreType.DMA,
                pltpu.SemaphoreType.DMA,
                pltpu.SemaphoreType.DMA,
            ],
            interpret=interpret,
        )(hL, hR, src3, dst3)

    return wrapped


BR = 1000
N_BLOCKS = N_NODES // BR
BN_SCALE = 1.0 / math.sqrt(1.0 + 1e-5)


def _mlp_tail(z, W2, b2, gm, bt, outL, outR):
    a = jnp.maximum(z, 0.0)
    z2 = jnp.dot(a, W2[...], preferred_element_type=jnp.float32) + b2[...]
    h = z2 * (BN_SCALE * gm[...]) + bt[...]
    h = jnp.maximum(h, 0.0)
    outL[...] = h[:, :128]
    outR[...] = h[:, 128:]


def _mlp_body(hL, hR, aL, aR, W1, b1, W2, b2, gm, bt, outL, outR):
    w1 = W1[...]
    z = (jnp.dot(hL[...] + aL[...], w1[:128, :],
                 preferred_element_type=jnp.float32)
         + jnp.dot(hR[...] + aR[...], w1[128:, :],
                   preferred_element_type=jnp.float32)
         + b1[...])
    _mlp_tail(z, W2, b2, gm, bt, outL, outR)


def _mlp0_body(x, aA, aB, W1, b1, W2, b2, gm, bt, outL, outR):
    u = x[...] + aA[...] + aB[...]
    z = jnp.dot(u, W1[...], preferred_element_type=jnp.float32) + b1[...]
    _mlp_tail(z, W2, b2, gm, bt, outL, outR)


def _build_mlp(first, interpret=False):
    din = 128 if first else 256
    return pl.pallas_call(
        _mlp0_body if first else _mlp_body,
        grid=(N_BLOCKS,),
        in_specs=[
            pl.BlockSpec((BR, 128), lambda i: (i, 0)),
            pl.BlockSpec((BR, 128), lambda i: (i, 0)),
            pl.BlockSpec((BR, 128), lambda i: (i, 0)),
            pl.BlockSpec((BR, 128), lambda i: (i, 0)),
            pl.BlockSpec((din, 256), lambda i: (0, 0)),
            pl.BlockSpec((1, 256), lambda i: (0, 0)),
            pl.BlockSpec((256, 256), lambda i: (0, 0)),
            pl.BlockSpec((1, 256), lambda i: (0, 0)),
            pl.BlockSpec((1, 256), lambda i: (0, 0)),
            pl.BlockSpec((1, 256), lambda i: (0, 0)),
        ][0 if not first else 1:],
        out_specs=[
            pl.BlockSpec((BR, 128), lambda i: (i, 0)),
            pl.BlockSpec((BR, 128), lambda i: (i, 0)),
        ],
        out_shape=[jax.ShapeDtypeStruct((N_NODES, 128), jnp.float32),
                   jax.ShapeDtypeStruct((N_NODES, 128), jnp.float32)],
        interpret=interpret,
    )


def _final_body(h0L, h0R, h1L, h1R, h2L, h2R, batch, Wj, bj, Wc1, bc1,
                Wc2, bc2, out, acc, cnt):
    i = pl.program_id(0)

    @pl.when(i == 0)
    def _():
        acc[...] = jnp.zeros_like(acc)
        cnt[...] = jnp.zeros_like(cnt)

    hcat = jnp.concatenate(
        [h0L[...], h0R[...], h1L[...], h1R[...], h2L[...], h2R[...]], axis=1)
    seg = jax.lax.broadcasted_iota(jnp.int32, (1, G_GRAPHS), 1)
    onehot = (batch[...] == seg).astype(jnp.float32)
    acc[...] += lax.dot_general(onehot, hcat,
                                (((0,), (0,)), ((), ())),
                                preferred_element_type=jnp.float32)
    ones = jnp.ones((BR, 1), jnp.float32)
    cnt[...] += lax.dot_general(onehot, ones,
                                (((0,), (0,)), ((), ())),
                                preferred_element_type=jnp.float32)

    @pl.when(i == N_BLOCKS - 1)
    def _():
        pooled = (jnp.dot(acc[...], Wj[...],
                          preferred_element_type=jnp.float32)
                  + cnt[...] * bj[...])
        q = jnp.maximum(
            jnp.dot(pooled, Wc1[...], preferred_element_type=jnp.float32)
            + bc1[...], 0.0)
        out[...] = (jnp.dot(q, Wc2[...], preferred_element_type=jnp.float32)
                    + bc2[...])


def _build_final(interpret=False):
    hspec = pl.BlockSpec((BR, 128), lambda i: (i, 0))
    return pl.pallas_call(
        _final_body,
        grid=(N_BLOCKS,),
        in_specs=[
            hspec, hspec, hspec, hspec, hspec, hspec,
            pl.BlockSpec((BR, 1), lambda i: (i, 0)),
            pl.BlockSpec((768, 256), lambda i: (0, 0)),
            pl.BlockSpec((1, 256), lambda i: (0, 0)),
            pl.BlockSpec((256, 128), lambda i: (0, 0)),
            pl.BlockSpec((1, 128), lambda i: (0, 0)),
            pl.BlockSpec((128, 128), lambda i: (0, 0)),
            pl.BlockSpec((1, 128), lambda i: (0, 0)),
        ],
        out_specs=pl.BlockSpec((G_GRAPHS, 128), lambda i: (0, 0)),
        out_shape=jax.ShapeDtypeStruct((G_GRAPHS, 128), jnp.float32),
        scratch_shapes=[
            pltpu.VMEM((G_GRAPHS, 768), jnp.float32),
            pltpu.VMEM((G_GRAPHS, 1), jnp.float32),
        ],
        interpret=interpret,
    )


def _run(x, edge_index, batch, params, jump, head, *, interpret=False):
    src3s = edge_index[0].reshape(TILES, -1, 1, CHUNK)
    dst3s = edge_index[1].reshape(TILES, -1, 1, CHUNK)
    src3f = edge_index[0].reshape(2 * TILES, -1, 1, CHUNK)
    dst3f = edge_index[1].reshape(2 * TILES, -1, 1, CHUNK)
    batch2 = batch.reshape(N_NODES, 1)

    sc_full = _build_sc_agg(False, interpret=interpret)
    sc_split = _build_sc_agg(True, interpret=interpret)
    mlp0 = _build_mlp(True, interpret=interpret)
    mlp = _build_mlp(False, interpret=interpret)
    fin = _build_final(interpret=interpret)

    def row(v):
        return v.reshape(1, -1)

    halves = []
    hL = hR = None
    for li, (W1, b1, W2, b2, gm, bt) in enumerate(params):
        if li == 0:
            aA, aB = sc_full(x, x, src3f, dst3f)
            hL, hR = mlp0(x, aA, aB, W1, row(b1), W2, row(b2),
                          row(gm), row(bt))
        else:
            aL, aR = sc_split(hL, hR, src3s, dst3s)
            hL, hR = mlp(hL, hR, aL, aR, W1, row(b1), W2, row(b2),
                         row(gm), row(bt))
        halves.extend([hL, hR])

    Wj, bj = jump
    Wc1, bc1, Wc2, bc2 = head
    Wc2p = jnp.pad(Wc2, ((0, 0), (0, 128 - Wc2.shape[1])))
    bc2p = jnp.pad(bc2, (0, 128 - bc2.shape[0]))
    outp = fin(*halves, batch2, Wj, row(bj), Wc1, row(bc1), Wc2p, row(bc2p))
    return outp[:, :Wc2.shape[1]]


def kernel(x, edge_index, batch, W1_0, b1_0, W2_0, b2_0, gamma_0, beta_0,
           W1_1, b1_1, W2_1, b2_1, gamma_1, beta_1,
           W1_2, b1_2, W2_2, b2_2, gamma_2, beta_2,
           Wj, bj, Wc1, bc1, Wc2, bc2):
    params = [
        (W1_0, b1_0, W2_0, b2_0, gamma_0, beta_0),
        (W1_1, b1_1, W2_1, b2_1, gamma_1, beta_1),
        (W1_2, b1_2, W2_2, b2_2, gamma_2, beta_2),
    ]
    return _run(x, edge_index, batch, params, (Wj, bj), (Wc1, bc1, Wc2, bc2))
